# baseline (speedup 1.0000x reference)
import functools

import jax
import jax.numpy as jnp
import numpy as np
from jax import lax
from jax.experimental import pallas as pl
from jax.experimental.pallas import tpu as pltpu
from jax.experimental.pallas import tpu_sc as plsc

N_PTS = 1048576
UV_RES = 1024
NW = 32
PW = N_PTS // NW
C = 2048
SUBS = PW // C
G = C // 16

_MAGIC = 8388608.0

N_FACES_PAD = 200192
N_VERTS_PAD = 100352
FCH = N_FACES_PAD // 16
VCH = N_VERTS_PAD // 16

_mesh = plsc.VectorSubcoreMesh(core_axis_name="c", subcore_axis_name="s")

_IDX = pltpu.VMEM((C,), jnp.int32)
_F32 = pltpu.VMEM((C,), jnp.float32)
_FSH = pltpu.VMEM_SHARED((N_FACES_PAD,), jnp.int32)
_VSH = pltpu.VMEM_SHARED((N_VERTS_PAD,), jnp.float32)

_DN = lax.GatherDimensionNumbers(offset_dims=(), collapsed_slice_dims=(0,),
                                 start_index_map=(0,))


def _dg(x, idx):
    return lax.gather(x, idx[:, None], _DN, (1,),
                      mode=lax.GatherScatterMode.PROMISE_IN_BOUNDS)


@functools.partial(
    pl.kernel,
    out_type=jax.ShapeDtypeStruct((3 * N_PTS,), jnp.float32),
    mesh=_mesh,
    scratch_types=[
        pltpu.VMEM((2 * C,), jnp.float32),
        pltpu.VMEM((2 * C,), jnp.float32),
        pltpu.VMEM((16,), jnp.float32),
        pltpu.VMEM((16,), jnp.float32),
        _IDX, _IDX,
        _IDX, _IDX,
        _IDX, _IDX, _IDX,
        _F32, _F32, _F32, _F32, _F32, _F32,
        _F32, _F32, _F32,
        _F32, _F32, _F32,
        _F32, _F32, _F32,
        pltpu.VMEM((3 * C,), jnp.float32),
        _FSH, _FSH, _FSH,
        _VSH, _VSH,
        _VSH, _VSH, _VSH,
        pltpu.SemaphoreType.DMA,
        pltpu.SemaphoreType.DMA,
        pltpu.SemaphoreType.DMA,
        pltpu.SemaphoreType.DMA,
    ],
)
def _uvto3d_sc(uvf_h, sx_h, sy_h, finds_h, f0_h, f1_h, f2_h,
               vux_h, vuy_h, v3x_h, v3y_h, v3z_h, out3_h,
               uvA_v, uvB_v, sx_v, sy_v, pixA_v, pixB_v,
               fidxA_v, fidxB_v, vid0_v, vid1_v, vid2_v,
               ax_v, ay_v, bx_v, by_v, cx_v, cy_v,
               pax_v, pay_v, paz_v, pbx_v, pby_v, pbz_v,
               pcx_v, pcy_v, pcz_v, out3_v,
               f0_s, f1_s, f2_s, vux_s, vuy_s, v3x_s, v3y_s, v3z_s,
               semg, sem1, semin, semout):
    sid = lax.axis_index("s")
    wid = sid * 2 + lax.axis_index("c")
    pltpu.sync_copy(sx_h, sx_v)
    pltpu.sync_copy(sy_h, sy_v)

    stage = [(f0_h, f0_s, FCH, pixA_v), (f1_h, f1_s, FCH, pixA_v),
             (f2_h, f2_s, FCH, pixA_v),
             (vux_h, vux_s, VCH, out3_v), (vuy_h, vuy_s, VCH, out3_v),
             (v3x_h, v3x_s, VCH, out3_v), (v3y_h, v3y_s, VCH, out3_v),
             (v3z_h, v3z_s, VCH, out3_v)]
    for src, dst, ch, bounce in stage:
        off = 0
        while off < ch:
            piece = min(C, ch - off)
            pltpu.sync_copy(src.at[pl.ds(sid * ch + off, piece)],
                            bounce.at[pl.ds(0, piece)])
            pltpu.sync_copy(bounce.at[pl.ds(0, piece)],
                            dst.at[pl.ds(sid * ch + off, piece)])
            off += piece
    plsc.subcore_barrier()

    sx = sx_v[...]
    sy = sy_v[...]
    base0 = wid * PW

    io16 = lax.iota(jnp.int32, 16)
    lo8 = io16 < 8
    p_ev = jnp.where(lo8, io16 * 2, (io16 - 8) * 2 + 1)
    p_od = jnp.where(lo8, io16 * 2 + 1, (io16 - 8) * 2)
    t_pnt, t_cmp = [], []
    for m in range(3):
        x = io16 + 16 * m
        q = (x.astype(jnp.float32) * 0.33333334).astype(jnp.int32)
        t_pnt.append(q)
        t_cmp.append(x - q * 3)

    def deinter(uvb_ref, g):
        vecA = uvb_ref[pl.ds(g * 32, 16)]
        vecB = uvb_ref[pl.ds(g * 32 + 16, 16)]
        uu = jnp.where(lo8, _dg(vecA, p_ev), _dg(vecB, p_od))
        vv = jnp.where(lo8, _dg(vecA, p_od), _dg(vecB, p_ev))
        return uu, vv

    def quant(uvb_ref, pix_ref):
        def body(g, c2):
            uu, vv = deinter(uvb_ref, g)
            fx = (uu * sx + _MAGIC) - _MAGIC
            fy = (vv * sy + _MAGIC) - _MAGIC
            pix = fy.astype(jnp.int32) * UV_RES + fx.astype(jnp.int32)
            pix_ref[pl.ds(g * 16, 16)] = pix
            return c2
        lax.fori_loop(0, G, body, 0)

    def bary(uvb_ref):
        def body(g, c2):
            sl = pl.ds(g * 16, 16)
            uu, vv = deinter(uvb_ref, g)
            axx = ax_v[sl]; ayy = ay_v[sl]
            bxx = bx_v[sl]; byy = by_v[sl]
            cxx = cx_v[sl]; cyy = cy_v[sl]
            v0x = bxx - axx; v0y = byy - ayy
            v1x = cxx - axx; v1y = cyy - ayy
            v2x = uu - axx;  v2y = vv - ayy
            d00 = v0x * v0x + v0y * v0y
            d01 = v0x * v1x + v0y * v1y
            d11 = v1x * v1x + v1y * v1y
            d20 = v2x * v0x + v2y * v0y
            d21 = v2x * v1x + v2y * v1y
            den = d00 * d11 - d01 * d01 + 1e-12
            vb = (d11 * d20 - d01 * d21) / den
            wb = (d00 * d21 - d01 * d20) / den
            ub = 1.0 - vb - wb
            px = pax_v[sl] * ub + pbx_v[sl] * vb + pcx_v[sl] * wb
            py = pay_v[sl] * ub + pby_v[sl] * vb + pcy_v[sl] * wb
            pz = paz_v[sl] * ub + pbz_v[sl] * vb + pcz_v[sl] * wb
            for m in range(3):
                vec = jnp.where(t_cmp[m] == 0, _dg(px, t_pnt[m]),
                                jnp.where(t_cmp[m] == 1, _dg(py, t_pnt[m]),
                                          _dg(pz, t_pnt[m])))
                out3_v[pl.ds(g * 48 + 16 * m, 16)] = vec
            return c2
        lax.fori_loop(0, G, body, 0)

    bufs = [(uvA_v, pixA_v, fidxA_v), (uvB_v, pixB_v, fidxB_v)]

    pltpu.sync_copy(uvf_h.at[pl.ds(2 * base0, 2 * C)], uvA_v)
    quant(uvA_v, pixA_v)
    w1 = pltpu.async_copy(finds_h.at[pixA_v], fidxA_v, sem1)

    out_cp = None
    for t in range(SUBS):
        cuv, _, cfidx = bufs[t % 2]
        w1.wait()
        w2 = [pltpu.async_copy(f_s.at[cfidx], vid, semg)
              for f_s, vid in ((f0_s, vid0_v), (f1_s, vid1_v), (f2_s, vid2_v))]
        if t + 1 < SUBS:
            nuv, npix, nfidx = bufs[(t + 1) % 2]
            nbase = base0 + (t + 1) * C
            pltpu.async_copy(uvf_h.at[pl.ds(2 * nbase, 2 * C)], nuv,
                             semin).wait()
            quant(nuv, npix)
            w1 = pltpu.async_copy(finds_h.at[npix], nfidx, sem1)
        for cp in w2:
            cp.wait()
        gathers = (
            (vid0_v, vux_s, ax_v), (vid0_v, vuy_s, ay_v),
            (vid1_v, vux_s, bx_v), (vid1_v, vuy_s, by_v),
            (vid2_v, vux_s, cx_v), (vid2_v, vuy_s, cy_v),
            (vid0_v, v3x_s, pax_v), (vid0_v, v3y_s, pay_v), (vid0_v, v3z_s, paz_v),
            (vid1_v, v3x_s, pbx_v), (vid1_v, v3y_s, pby_v), (vid1_v, v3z_s, pbz_v),
            (vid2_v, v3x_s, pcx_v), (vid2_v, v3y_s, pcy_v), (vid2_v, v3z_s, pcz_v),
        )
        w3 = [pltpu.async_copy(tab.at[idx], dst, semg)
              for idx, tab, dst in gathers]
        if out_cp is not None:
            out_cp.wait()
        for cp in w3:
            cp.wait()
        bary(cuv)
        base = base0 + t * C
        out_cp = pltpu.async_copy(out3_v, out3_h.at[pl.ds(3 * base, 3 * C)],
                                  semout)
    out_cp.wait()


def kernel(uv, verts_uv, verts_3d, faces, face_inds, uv_map_size):
    n = uv.shape[0]
    uvf = uv.reshape(-1)
    sx = jnp.broadcast_to(uv_map_size[0, 0], (16,)).astype(jnp.float32)
    sy = jnp.broadcast_to(uv_map_size[0, 1], (16,)).astype(jnp.float32)
    finds = face_inds.reshape(-1)
    fpad = N_FACES_PAD - faces.shape[0]
    vpad = N_VERTS_PAD - verts_uv.shape[0]
    f0, f1, f2 = (jnp.pad(faces[:, i], (0, fpad)) for i in range(3))
    vux, vuy = (jnp.pad(verts_uv[:, i], (0, vpad)) for i in range(2))
    v3x, v3y, v3z = (jnp.pad(verts_3d[:, i], (0, vpad)) for i in range(3))
    out3 = _uvto3d_sc(uvf, sx, sy, finds, f0, f1, f2,
                      vux, vuy, v3x, v3y, v3z)
    return out3.reshape(n, 3)


# speedup vs baseline: 7.5301x; 7.5301x over previous
import functools

import jax
import jax.numpy as jnp
from jax import lax
from jax.experimental import pallas as pl
from jax.experimental.pallas import tpu as pltpu
from jax.experimental.pallas import tpu_sc as plsc

N_PTS = 1048576
UV_RES = 1024
NW = 32
PW = N_PTS // NW
C = 2048
SUBS = PW // C
G = C // 16

_MAGIC = 8388608.0

N_FACES_PAD = 200192
N_VERTS_PAD = 100352
FCH = N_FACES_PAD // 16
VCH = N_VERTS_PAD // 16

_mesh = plsc.VectorSubcoreMesh(core_axis_name="c", subcore_axis_name="s")

_IDX = pltpu.VMEM((C,), jnp.int32)
_F32 = pltpu.VMEM((C,), jnp.float32)
_FSH = pltpu.VMEM_SHARED((N_FACES_PAD,), jnp.int32)
_VSH = pltpu.VMEM_SHARED((N_VERTS_PAD,), jnp.float32)


@functools.partial(
    pl.kernel,
    out_type=(jax.ShapeDtypeStruct((N_PTS,), jnp.float32),
              jax.ShapeDtypeStruct((N_PTS,), jnp.float32),
              jax.ShapeDtypeStruct((N_PTS,), jnp.float32)),
    mesh=_mesh,
    scratch_types=[
        _F32, _F32, _F32, _F32,
        pltpu.VMEM((16,), jnp.float32),
        pltpu.VMEM((16,), jnp.float32),
        _IDX, _IDX,
        _IDX, _IDX,
        _IDX, _IDX, _IDX,
        _F32, _F32, _F32, _F32, _F32, _F32,
        _F32, _F32, _F32,
        _F32, _F32, _F32,
        _F32, _F32, _F32,
        _F32, _F32, _F32,
        _FSH, _FSH, _FSH,
        _VSH, _VSH,
        _VSH, _VSH, _VSH,
        pltpu.SemaphoreType.DMA,
        pltpu.SemaphoreType.DMA,
        pltpu.SemaphoreType.DMA,
        pltpu.SemaphoreType.DMA,
    ],
)
def _uvto3d_sc(u_h, v_h, sx_h, sy_h, finds_h, f0_h, f1_h, f2_h,
               vux_h, vuy_h, v3x_h, v3y_h, v3z_h, outx_h, outy_h, outz_h,
               uA_v, vA_v, uB_v, vB_v, sx_v, sy_v, pixA_v, pixB_v,
               fidxA_v, fidxB_v, vid0_v, vid1_v, vid2_v,
               ax_v, ay_v, bx_v, by_v, cx_v, cy_v,
               pax_v, pay_v, paz_v, pbx_v, pby_v, pbz_v,
               pcx_v, pcy_v, pcz_v, outx_v, outy_v, outz_v,
               f0_s, f1_s, f2_s, vux_s, vuy_s, v3x_s, v3y_s, v3z_s,
               semg, sem1, semin, semout):
    sid = lax.axis_index("s")
    wid = sid * 2 + lax.axis_index("c")
    pltpu.sync_copy(sx_h, sx_v)
    pltpu.sync_copy(sy_h, sy_v)

    stage = [(f0_h, f0_s, FCH, pixA_v), (f1_h, f1_s, FCH, pixA_v),
             (f2_h, f2_s, FCH, pixA_v),
             (vux_h, vux_s, VCH, outx_v), (vuy_h, vuy_s, VCH, outx_v),
             (v3x_h, v3x_s, VCH, outx_v), (v3y_h, v3y_s, VCH, outx_v),
             (v3z_h, v3z_s, VCH, outx_v)]
    for src, dst, ch, bounce in stage:
        off = 0
        while off < ch:
            piece = min(C, ch - off)
            pltpu.sync_copy(src.at[pl.ds(sid * ch + off, piece)],
                            bounce.at[pl.ds(0, piece)])
            pltpu.sync_copy(bounce.at[pl.ds(0, piece)],
                            dst.at[pl.ds(sid * ch + off, piece)])
            off += piece
    plsc.subcore_barrier()

    sx = sx_v[...]
    sy = sy_v[...]
    base0 = wid * PW

    def quant(u_ref, v_ref, pix_ref):
        def body(g, c2):
            uu = u_ref[pl.ds(g * 16, 16)]
            vv = v_ref[pl.ds(g * 16, 16)]
            fx = (uu * sx + _MAGIC) - _MAGIC
            fy = (vv * sy + _MAGIC) - _MAGIC
            pix = fy.astype(jnp.int32) * UV_RES + fx.astype(jnp.int32)
            pix_ref[pl.ds(g * 16, 16)] = pix
            return c2
        lax.fori_loop(0, G, body, 0)

    def bary(u_ref, v_ref):
        def body(g, c2):
            sl = pl.ds(g * 16, 16)
            uu = u_ref[sl]
            vv = v_ref[sl]
            axx = ax_v[sl]; ayy = ay_v[sl]
            bxx = bx_v[sl]; byy = by_v[sl]
            cxx = cx_v[sl]; cyy = cy_v[sl]
            v0x = bxx - axx; v0y = byy - ayy
            v1x = cxx - axx; v1y = cyy - ayy
            v2x = uu - axx;  v2y = vv - ayy
            d00 = v0x * v0x + v0y * v0y
            d01 = v0x * v1x + v0y * v1y
            d11 = v1x * v1x + v1y * v1y
            d20 = v2x * v0x + v2y * v0y
            d21 = v2x * v1x + v2y * v1y
            den = d00 * d11 - d01 * d01 + 1e-12
            vb = (d11 * d20 - d01 * d21) / den
            wb = (d00 * d21 - d01 * d20) / den
            ub = 1.0 - vb - wb
            outx_v[sl] = pax_v[sl] * ub + pbx_v[sl] * vb + pcx_v[sl] * wb
            outy_v[sl] = pay_v[sl] * ub + pby_v[sl] * vb + pcy_v[sl] * wb
            outz_v[sl] = paz_v[sl] * ub + pbz_v[sl] * vb + pcz_v[sl] * wb
            return c2
        lax.fori_loop(0, G, body, 0)

    bufs = [(uA_v, vA_v, pixA_v, fidxA_v), (uB_v, vB_v, pixB_v, fidxB_v)]

    pltpu.sync_copy(u_h.at[pl.ds(base0, C)], uA_v)
    pltpu.sync_copy(v_h.at[pl.ds(base0, C)], vA_v)
    quant(uA_v, vA_v, pixA_v)
    w1 = pltpu.async_copy(finds_h.at[pixA_v], fidxA_v, sem1)

    out_cps = None
    for t in range(SUBS):
        cu, cv, _, cfidx = bufs[t % 2]
        w1.wait()
        w2 = [pltpu.async_copy(f_s.at[cfidx], vid, semg)
              for f_s, vid in ((f0_s, vid0_v), (f1_s, vid1_v), (f2_s, vid2_v))]
        if t + 1 < SUBS:
            nu, nv, npix, nfidx = bufs[(t + 1) % 2]
            nbase = base0 + (t + 1) * C
            i0 = pltpu.async_copy(u_h.at[pl.ds(nbase, C)], nu, semin)
            i1 = pltpu.async_copy(v_h.at[pl.ds(nbase, C)], nv, semin)
            i0.wait(); i1.wait()
            quant(nu, nv, npix)
            w1 = pltpu.async_copy(finds_h.at[npix], nfidx, sem1)
        for cp in w2:
            cp.wait()
        gathers = (
            (vid0_v, vux_s, ax_v), (vid0_v, vuy_s, ay_v),
            (vid1_v, vux_s, bx_v), (vid1_v, vuy_s, by_v),
            (vid2_v, vux_s, cx_v), (vid2_v, vuy_s, cy_v),
            (vid0_v, v3x_s, pax_v), (vid0_v, v3y_s, pay_v), (vid0_v, v3z_s, paz_v),
            (vid1_v, v3x_s, pbx_v), (vid1_v, v3y_s, pby_v), (vid1_v, v3z_s, pbz_v),
            (vid2_v, v3x_s, pcx_v), (vid2_v, v3y_s, pcy_v), (vid2_v, v3z_s, pcz_v),
        )
        w3 = [pltpu.async_copy(tab.at[idx], dst, semg)
              for idx, tab, dst in gathers]
        if out_cps is not None:
            for cp in out_cps:
                cp.wait()
        for cp in w3:
            cp.wait()
        bary(cu, cv)
        base = base0 + t * C
        out_cps = [pltpu.async_copy(outx_v, outx_h.at[pl.ds(base, C)], semout),
                   pltpu.async_copy(outy_v, outy_h.at[pl.ds(base, C)], semout),
                   pltpu.async_copy(outz_v, outz_h.at[pl.ds(base, C)], semout)]
    for cp in out_cps:
        cp.wait()


def kernel(uv, verts_uv, verts_3d, faces, face_inds, uv_map_size):
    n = uv.shape[0]
    u1 = uv[:, 0]
    v1 = uv[:, 1]
    sx = jnp.broadcast_to(uv_map_size[0, 0], (16,)).astype(jnp.float32)
    sy = jnp.broadcast_to(uv_map_size[0, 1], (16,)).astype(jnp.float32)
    finds = face_inds.reshape(-1)
    fpad = N_FACES_PAD - faces.shape[0]
    vpad = N_VERTS_PAD - verts_uv.shape[0]
    f0, f1, f2 = (jnp.pad(faces[:, i], (0, fpad)) for i in range(3))
    vux, vuy = (jnp.pad(verts_uv[:, i], (0, vpad)) for i in range(2))
    v3x, v3y, v3z = (jnp.pad(verts_3d[:, i], (0, vpad)) for i in range(3))
    ox, oy, oz = _uvto3d_sc(u1, v1, sx, sy, finds, f0, f1, f2,
                            vux, vuy, v3x, v3y, v3z)
    return jnp.stack([ox, oy, oz], axis=1)
